# baseline (speedup 1.0000x reference)
import functools

import jax
import jax.numpy as jnp
from jax import lax
from jax.experimental import pallas as pl
from jax.experimental.pallas import tpu as pltpu
from jax.experimental.pallas import tpu_sc as plsc

B, L, D = 4096, 50, 32
NC, NS, LANES = 2, 16, 16
NW = NC * NS
ROWS_W = B // NW
LOOK_W = ROWS_W * L
LC = 5
CHUNK = LC * ROWS_W
N_CHUNK = L // LC
VPR = ROWS_W // LANES
TBLOCK = 1024


def _t_body(src_ref, dst_ref):
    dst_ref[...] = src_ref[...].T.reshape(TBLOCK // 4, 4, D)


def _to_row_major(table_t):
    V = table_t.shape[1]
    grid = (V + TBLOCK - 1) // TBLOCK
    packed = pl.pallas_call(
        _t_body,
        grid=(grid,),
        in_specs=[pl.BlockSpec((D, TBLOCK), lambda i: (0, i))],
        out_specs=pl.BlockSpec((TBLOCK // 4, 4, D), lambda i: (i, 0, 0)),
        out_shape=jax.ShapeDtypeStruct((V // 4, 4, D), jnp.float32),
    )(table_t)
    return packed.reshape(V, D)


def _o_body(src_ref, dst_ref):
    x = src_ref[...]
    y = x.reshape(L, ROWS_W, D).transpose(0, 2, 1)
    dst_ref[...] = y.reshape(L, 4, 1, 8, ROWS_W)


def _to_out_layout(emb_wl):
    packed = emb_wl.reshape(B * L // 4, 4, D)
    return pl.pallas_call(
        _o_body,
        grid=(NW,),
        in_specs=[pl.BlockSpec((LOOK_W // 4, 4, D), lambda w: (w, 0, 0))],
        out_specs=pl.BlockSpec((L, 4, 1, 8, ROWS_W), lambda w: (0, 0, w, 0, 0)),
        out_shape=jax.ShapeDtypeStruct((L, 4, NW, 8, ROWS_W), jnp.float32),
    )(packed)


def _sc_body(tab_i, tab_c, tab_t, idx_i, idx_c, idx_t,
             out_i, out_c, out_t, sl_i, sl_c, sl_t,
             idx_f, rows_v, slen_v, sem):
    wid = lax.axis_index("s") * NC + lax.axis_index("c")
    base = wid * LOOK_W

    for tab, idx_h, out_h, sl_h in ((tab_i, idx_i, out_i, sl_i),
                                    (tab_c, idx_c, out_c, sl_c),
                                    (tab_t, idx_t, out_t, sl_t)):
        pltpu.sync_copy(idx_h.at[pl.ds(base, LOOK_W)], idx_f)

        def chunk_body(c, _, tab=tab, out_h=out_h):
            pltpu.async_copy(
                tab.at[idx_f.at[pl.ds(c * CHUNK, CHUNK)]],
                rows_v, sem).wait()
            pltpu.sync_copy(rows_v, out_h.at[pl.ds(base + c * CHUNK, CHUNK)])
            return _
        lax.fori_loop(0, N_CHUNK, chunk_body, 0)

        acc = [jnp.zeros((LANES,), jnp.int32) for _ in range(VPR)]
        for l in range(L):
            for j in range(VPR):
                v = idx_f[pl.ds(l * ROWS_W + j * LANES, LANES)]
                acc[j] = acc[j] + jnp.where(v != 0, 1, 0).astype(jnp.int32)
        for j in range(VPR):
            slen_v[pl.ds(j * LANES, LANES)] = acc[j]
        pltpu.sync_copy(slen_v, sl_h.at[pl.ds(wid * ROWS_W, ROWS_W)])


@jax.jit
def _run(item_hist, cate_hist, user_tags, table_item, table_cate, table_tags):
    idx1d = lambda a: a.reshape(NW, ROWS_W, L).transpose(0, 2, 1).reshape(-1)

    tabs = [_to_row_major(t.T) for t in (table_item, table_cate, table_tags)]

    mesh = plsc.VectorSubcoreMesh(core_axis_name="c", subcore_axis_name="s")
    ewl = jax.ShapeDtypeStruct((B * L, D), jnp.float32)
    f = pl.kernel(
        _sc_body,
        out_type=(ewl, ewl, ewl,
                  jax.ShapeDtypeStruct((B,), jnp.int32),
                  jax.ShapeDtypeStruct((B,), jnp.int32),
                  jax.ShapeDtypeStruct((B,), jnp.int32)),
        mesh=mesh,
        compiler_params=pltpu.CompilerParams(use_tc_tiling_on_sc=False),
        scratch_types=[
            pltpu.VMEM((LOOK_W,), jnp.int32),
            pltpu.VMEM((CHUNK, D), jnp.float32),
            pltpu.VMEM((ROWS_W,), jnp.int32),
            pltpu.SemaphoreType.DMA,
        ],
    )
    e_i, e_c, e_t, sl_i, sl_c, sl_t = f(
        tabs[0], tabs[1], tabs[2],
        idx1d(item_hist), idx1d(cate_hist), idx1d(user_tags),
    )
    def unbit(e):
        o5 = _to_out_layout(e)
        return o5.transpose(2, 4, 0, 1, 3).reshape(B, L, D)
    return (unbit(e_i), unbit(e_c), unbit(e_t), sl_i, sl_c, sl_t)


def kernel(item_hist, cate_hist, user_tags, table_item, table_cate, table_tags):
    return _run(item_hist, cate_hist, user_tags,
                table_item, table_cate, table_tags)


# speedup vs baseline: 1.1030x; 1.1030x over previous
import functools

import jax
import jax.numpy as jnp
from jax import lax
from jax.experimental import pallas as pl
from jax.experimental.pallas import tpu as pltpu
from jax.experimental.pallas import tpu_sc as plsc

B, L, D = 4096, 50, 32
NC, NS, LANES = 2, 16, 16
NW = NC * NS
ROWS_W = B // NW
LOOK_W = ROWS_W * L
LC = 5
CHUNK = LC * ROWS_W
N_CHUNK = L // LC
VPR = ROWS_W // LANES
TBLOCK = 1024


def _t_body(src_ref, dst_ref):
    y = src_ref[...].T.reshape(TBLOCK // 4, 4, D)
    dst_ref[...] = jnp.concatenate([y[:, a, :] for a in range(4)], axis=1)


def _to_row_major(table_t):
    V = table_t.shape[1]
    grid = (V + TBLOCK - 1) // TBLOCK
    packed = pl.pallas_call(
        _t_body,
        grid=(grid,),
        in_specs=[pl.BlockSpec((D, TBLOCK), lambda i: (0, i))],
        out_specs=pl.BlockSpec((TBLOCK // 4, 4 * D), lambda i: (i, 0)),
        out_shape=jax.ShapeDtypeStruct((V // 4, 4 * D), jnp.float32),
    )(table_t)
    return packed.reshape(V, D)


def _o_body(src_ref, dst_ref):
    x = src_ref[...]
    y = jnp.stack([x[:, 32 * a:32 * a + 32] for a in range(4)], axis=1)
    rows = y.reshape(L, ROWS_W, D)
    z = rows.transpose(0, 2, 1)
    dst_ref[...] = z.reshape(L, 4, 1, 8, ROWS_W)


def _to_out_layout(emb_wl):
    packed = emb_wl.reshape(B * L // 4, 4 * D)
    return pl.pallas_call(
        _o_body,
        grid=(NW,),
        in_specs=[pl.BlockSpec((LOOK_W // 4, 4 * D), lambda w: (w, 0))],
        out_specs=pl.BlockSpec((L, 4, 1, 8, ROWS_W), lambda w: (0, 0, w, 0, 0)),
        out_shape=jax.ShapeDtypeStruct((L, 4, NW, 8, ROWS_W), jnp.float32),
    )(packed)


def _sc_body(tab_i, tab_c, tab_t, idx_i, idx_c, idx_t,
             out_i, out_c, out_t, sl_i, sl_c, sl_t,
             idx_f, rows_v, slen_v, sem):
    wid = lax.axis_index("s") * NC + lax.axis_index("c")
    base = wid * LOOK_W

    for tab, idx_h, out_h, sl_h in ((tab_i, idx_i, out_i, sl_i),
                                    (tab_c, idx_c, out_c, sl_c),
                                    (tab_t, idx_t, out_t, sl_t)):
        pltpu.sync_copy(idx_h.at[pl.ds(base, LOOK_W)], idx_f)

        def chunk_body(c, _, tab=tab, out_h=out_h):
            pltpu.async_copy(
                tab.at[idx_f.at[pl.ds(c * CHUNK, CHUNK)]],
                rows_v, sem).wait()
            pltpu.sync_copy(rows_v, out_h.at[pl.ds(base + c * CHUNK, CHUNK)])
            return _
        lax.fori_loop(0, N_CHUNK, chunk_body, 0)

        acc = [jnp.zeros((LANES,), jnp.int32) for _ in range(VPR)]
        for l in range(L):
            for j in range(VPR):
                v = idx_f[pl.ds(l * ROWS_W + j * LANES, LANES)]
                acc[j] = acc[j] + jnp.where(v != 0, 1, 0).astype(jnp.int32)
        for j in range(VPR):
            slen_v[pl.ds(j * LANES, LANES)] = acc[j]
        pltpu.sync_copy(slen_v, sl_h.at[pl.ds(wid * ROWS_W, ROWS_W)])


@jax.jit
def _run(item_hist, cate_hist, user_tags, table_item, table_cate, table_tags):
    idx1d = lambda a: a.reshape(NW, ROWS_W, L).transpose(0, 2, 1).reshape(-1)

    tabs = [_to_row_major(t.T) for t in (table_item, table_cate, table_tags)]

    mesh = plsc.VectorSubcoreMesh(core_axis_name="c", subcore_axis_name="s")
    ewl = jax.ShapeDtypeStruct((B * L, D), jnp.float32)
    f = pl.kernel(
        _sc_body,
        out_type=(ewl, ewl, ewl,
                  jax.ShapeDtypeStruct((B,), jnp.int32),
                  jax.ShapeDtypeStruct((B,), jnp.int32),
                  jax.ShapeDtypeStruct((B,), jnp.int32)),
        mesh=mesh,
        compiler_params=pltpu.CompilerParams(use_tc_tiling_on_sc=False),
        scratch_types=[
            pltpu.VMEM((LOOK_W,), jnp.int32),
            pltpu.VMEM((CHUNK, D), jnp.float32),
            pltpu.VMEM((ROWS_W,), jnp.int32),
            pltpu.SemaphoreType.DMA,
        ],
    )
    e_i, e_c, e_t, sl_i, sl_c, sl_t = f(
        tabs[0], tabs[1], tabs[2],
        idx1d(item_hist), idx1d(cate_hist), idx1d(user_tags),
    )
    def unbit(e):
        o5 = _to_out_layout(e)
        return o5.transpose(2, 4, 0, 1, 3).reshape(B, L, D)
    return (unbit(e_i), unbit(e_c), unbit(e_t), sl_i, sl_c, sl_t)


def kernel(item_hist, cate_hist, user_tags, table_item, table_cate, table_tags):
    return _run(item_hist, cate_hist, user_tags,
                table_item, table_cate, table_tags)


# speedup vs baseline: 1.1146x; 1.0105x over previous
import functools

import jax
import jax.numpy as jnp
from jax import lax
from jax.experimental import pallas as pl
from jax.experimental.pallas import tpu as pltpu
from jax.experimental.pallas import tpu_sc as plsc

B, L, D = 4096, 50, 32
NC, NS, LANES = 2, 16, 16
NW = NC * NS
ROWS_W = B // NW
LOOK_W = ROWS_W * L
LC = 5
CHUNK = LC * ROWS_W
N_CHUNK = L // LC
VPR = ROWS_W // LANES
TBLOCK = 1024


def _t_body(src_ref, dst_ref):
    x = src_ref[...]
    eye = jnp.eye(D, dtype=jnp.float32)
    y = lax.dot_general(x, eye, (((0,), (0,)), ((), ())),
                        preferred_element_type=jnp.float32)
    y3 = y.reshape(TBLOCK // 4, 4, D)
    dst_ref[...] = jnp.concatenate([y3[:, a, :] for a in range(4)], axis=1)


def _to_row_major(table_t):
    V = table_t.shape[1]
    grid = (V + TBLOCK - 1) // TBLOCK
    packed = pl.pallas_call(
        _t_body,
        grid=(grid,),
        in_specs=[pl.BlockSpec((D, TBLOCK), lambda i: (0, i))],
        out_specs=pl.BlockSpec((TBLOCK // 4, 4 * D), lambda i: (i, 0)),
        out_shape=jax.ShapeDtypeStruct((V // 4, 4 * D), jnp.float32),
    )(table_t)
    return packed.reshape(V, D)


def _o_body(src_ref, dst_ref):
    x = src_ref[...]
    y = jnp.stack([x[:, 32 * a:32 * a + 32] for a in range(4)], axis=1)
    rows = y.reshape(L, ROWS_W, D)
    eye = jnp.eye(D, dtype=jnp.float32)
    z = lax.dot_general(eye, rows, (((1,), (2,)), ((), ())),
                        preferred_element_type=jnp.float32)
    dst_ref[...] = z.transpose(1, 0, 2).reshape(L, 4, 1, 8, ROWS_W)


def _to_out_layout(emb_wl):
    packed = emb_wl.reshape(B * L // 4, 4 * D)
    return pl.pallas_call(
        _o_body,
        grid=(NW,),
        in_specs=[pl.BlockSpec((LOOK_W // 4, 4 * D), lambda w: (w, 0))],
        out_specs=pl.BlockSpec((L, 4, 1, 8, ROWS_W), lambda w: (0, 0, w, 0, 0)),
        out_shape=jax.ShapeDtypeStruct((L, 4, NW, 8, ROWS_W), jnp.float32),
    )(packed)


def _sc_body(tab_i, tab_c, tab_t, idx_i, idx_c, idx_t,
             out_i, out_c, out_t, sl_i, sl_c, sl_t,
             idx_f, rows_v, slen_v, sem):
    wid = lax.axis_index("s") * NC + lax.axis_index("c")
    base = wid * LOOK_W

    for tab, idx_h, out_h, sl_h in ((tab_i, idx_i, out_i, sl_i),
                                    (tab_c, idx_c, out_c, sl_c),
                                    (tab_t, idx_t, out_t, sl_t)):
        pltpu.sync_copy(idx_h.at[pl.ds(base, LOOK_W)], idx_f)

        def chunk_body(c, _, tab=tab, out_h=out_h):
            pltpu.async_copy(
                tab.at[idx_f.at[pl.ds(c * CHUNK, CHUNK)]],
                rows_v, sem).wait()
            pltpu.sync_copy(rows_v, out_h.at[pl.ds(base + c * CHUNK, CHUNK)])
            return _
        lax.fori_loop(0, N_CHUNK, chunk_body, 0)

        acc = [jnp.zeros((LANES,), jnp.int32) for _ in range(VPR)]
        for l in range(L):
            for j in range(VPR):
                v = idx_f[pl.ds(l * ROWS_W + j * LANES, LANES)]
                acc[j] = acc[j] + jnp.where(v != 0, 1, 0).astype(jnp.int32)
        for j in range(VPR):
            slen_v[pl.ds(j * LANES, LANES)] = acc[j]
        pltpu.sync_copy(slen_v, sl_h.at[pl.ds(wid * ROWS_W, ROWS_W)])


@jax.jit
def _run(item_hist, cate_hist, user_tags, table_item, table_cate, table_tags):
    idx1d = lambda a: a.reshape(NW, ROWS_W, L).transpose(0, 2, 1).reshape(-1)

    tabs = [_to_row_major(t.T) for t in (table_item, table_cate, table_tags)]

    mesh = plsc.VectorSubcoreMesh(core_axis_name="c", subcore_axis_name="s")
    ewl = jax.ShapeDtypeStruct((B * L, D), jnp.float32)
    f = pl.kernel(
        _sc_body,
        out_type=(ewl, ewl, ewl,
                  jax.ShapeDtypeStruct((B,), jnp.int32),
                  jax.ShapeDtypeStruct((B,), jnp.int32),
                  jax.ShapeDtypeStruct((B,), jnp.int32)),
        mesh=mesh,
        compiler_params=pltpu.CompilerParams(use_tc_tiling_on_sc=False),
        scratch_types=[
            pltpu.VMEM((LOOK_W,), jnp.int32),
            pltpu.VMEM((CHUNK, D), jnp.float32),
            pltpu.VMEM((ROWS_W,), jnp.int32),
            pltpu.SemaphoreType.DMA,
        ],
    )
    e_i, e_c, e_t, sl_i, sl_c, sl_t = f(
        tabs[0], tabs[1], tabs[2],
        idx1d(item_hist), idx1d(cate_hist), idx1d(user_tags),
    )
    def unbit(e):
        o5 = _to_out_layout(e)
        return o5.transpose(2, 4, 0, 1, 3).reshape(B, L, D)
    return (unbit(e_i), unbit(e_c), unbit(e_t), sl_i, sl_c, sl_t)


def kernel(item_hist, cate_hist, user_tags, table_item, table_cate, table_tags):
    return _run(item_hist, cate_hist, user_tags,
                table_item, table_cate, table_tags)


# speedup vs baseline: 1.7991x; 1.6142x over previous
import functools

import jax
import jax.numpy as jnp
from jax import lax
from jax.experimental import pallas as pl
from jax.experimental.pallas import tpu as pltpu
from jax.experimental.pallas import tpu_sc as plsc

B, L, D = 4096, 50, 32
NC, NS, LANES = 2, 16, 16
NW = NC * NS
ROWS_W = B // NW
LOOK_W = ROWS_W * L
LC = 5
CHUNK = LC * ROWS_W
N_CHUNK = L // LC
VPR = ROWS_W // LANES


def _sc_body(tab_i, tab_c, tab_t, idx_i, idx_c, idx_t,
             out_i, out_c, out_t, sl_i, sl_c, sl_t,
             idx_f, rows_v, slen_v, sem):
    wid = lax.axis_index("s") * NC + lax.axis_index("c")
    base = wid * LOOK_W

    for tab, idx_h, out_h, sl_h in ((tab_i, idx_i, out_i, sl_i),
                                    (tab_c, idx_c, out_c, sl_c),
                                    (tab_t, idx_t, out_t, sl_t)):
        pltpu.sync_copy(idx_h.at[pl.ds(base, LOOK_W)], idx_f)

        def chunk_body(c, _, tab=tab, out_h=out_h):
            pltpu.async_copy(
                tab.at[idx_f.at[pl.ds(c * CHUNK, CHUNK)]],
                rows_v, sem).wait()
            pltpu.sync_copy(rows_v, out_h.at[pl.ds(base + c * CHUNK, CHUNK)])
            return _
        lax.fori_loop(0, N_CHUNK, chunk_body, 0)

        acc = [jnp.zeros((LANES,), jnp.int32) for _ in range(VPR)]
        for l in range(L):
            for j in range(VPR):
                v = idx_f[pl.ds(l * ROWS_W + j * LANES, LANES)]
                acc[j] = acc[j] + jnp.where(v != 0, 1, 0).astype(jnp.int32)
        for j in range(VPR):
            slen_v[pl.ds(j * LANES, LANES)] = acc[j]
        pltpu.sync_copy(slen_v, sl_h.at[pl.ds(wid * ROWS_W, ROWS_W)])


@jax.jit
def _run(item_hist, cate_hist, user_tags, table_item, table_cate, table_tags):
    idx1d = lambda a: a.reshape(NW, ROWS_W, L).transpose(0, 2, 1).reshape(-1)

    mesh = plsc.VectorSubcoreMesh(core_axis_name="c", subcore_axis_name="s")
    ewl = jax.ShapeDtypeStruct((B * L, D), jnp.float32)
    f = pl.kernel(
        _sc_body,
        out_type=(ewl, ewl, ewl,
                  jax.ShapeDtypeStruct((B,), jnp.int32),
                  jax.ShapeDtypeStruct((B,), jnp.int32),
                  jax.ShapeDtypeStruct((B,), jnp.int32)),
        mesh=mesh,
        compiler_params=pltpu.CompilerParams(use_tc_tiling_on_sc=False),
        scratch_types=[
            pltpu.VMEM((LOOK_W,), jnp.int32),
            pltpu.VMEM((CHUNK, D), jnp.float32),
            pltpu.VMEM((ROWS_W,), jnp.int32),
            pltpu.SemaphoreType.DMA,
        ],
    )
    e_i, e_c, e_t, sl_i, sl_c, sl_t = f(
        table_item, table_cate, table_tags,
        idx1d(item_hist), idx1d(cate_hist), idx1d(user_tags),
    )
    unblk = lambda e: (e.reshape(NW, L, ROWS_W, D)
                       .transpose(0, 2, 1, 3).reshape(B, L, D))
    return (unblk(e_i), unblk(e_c), unblk(e_t), sl_i, sl_c, sl_t)


def kernel(item_hist, cate_hist, user_tags, table_item, table_cate, table_tags):
    return _run(item_hist, cate_hist, user_tags,
                table_item, table_cate, table_tags)
